# Initial kernel scaffold; baseline (speedup 1.0000x reference)
import jax
import jax.numpy as jnp
from jax.experimental import pallas as pl


def kernel(input, nodes, clicked):
    raise NotImplementedError("write your pallas kernel here")


# speedup vs baseline: 152.2736x; 152.2736x over previous
import functools

import jax
import jax.numpy as jnp
from jax import lax
from jax.experimental import pallas as pl
from jax.experimental.pallas import tpu as pltpu
from jax.experimental.pallas import tpu_sc as plsc

N_NODES = 100000
N_EDGES = 3200000
N_CLICKED = 64
GAMMA = 0.5
ALPHA = 0.25
EPS = 1e-6

NPAD = 100352
ROWS = 784
LANES = 128
NUM_CORES = 2
NUM_SUBCORES = 16
NW = NUM_CORES * NUM_SUBCORES
PER_TILE = N_EDGES // NW
W = 2000
N_WINDOWS = PER_TILE // W
POS_W = 1 << 24


def _sc_scatter_body(edge_hbm, zeros_hbm, ones_hbm, posw_hbm, clicked_hbm,
                     out_hbm, idx_v, ones_v, clk_v, posw_v, cnt_s):
    c = lax.axis_index("c")
    s = lax.axis_index("s")
    wid = c * NUM_SUBCORES + s

    pltpu.sync_copy(ones_hbm, ones_v)

    @pl.when(s == 0)
    def _init():
        pltpu.sync_copy(zeros_hbm, cnt_s)

    plsc.subcore_barrier()

    base = wid * PER_TILE

    def body(w, carry):
        pltpu.sync_copy(edge_hbm.at[pl.ds(base + w * W, W)], idx_v)
        pltpu.sync_copy(ones_v, cnt_s.at[idx_v], add=True)
        return carry

    lax.fori_loop(0, N_WINDOWS, body, 0)

    @pl.when(wid == 0)
    def _clicked():
        pltpu.sync_copy(clicked_hbm, clk_v)
        pltpu.sync_copy(posw_hbm, posw_v)
        pltpu.sync_copy(posw_v, cnt_s.at[clk_v], add=True)

    plsc.subcore_barrier()

    @pl.when(s == 0)
    def _flush():
        pltpu.sync_copy(cnt_s, out_hbm.at[c])


_sc_scatter = pl.kernel(
    _sc_scatter_body,
    out_type=jax.ShapeDtypeStruct((NUM_CORES, NPAD), jnp.int32),
    mesh=plsc.VectorSubcoreMesh(core_axis_name="c", subcore_axis_name="s"),
    scratch_types=[
        pltpu.VMEM((W,), jnp.int32),
        pltpu.VMEM((W,), jnp.int32),
        pltpu.VMEM((N_CLICKED,), jnp.int32),
        pltpu.VMEM((N_CLICKED,), jnp.int32),
        pltpu.VMEM_SHARED((NPAD,), jnp.int32),
    ],
)


def _tc_focal_body(x_ref, cnt_ref, o_ref):
    cnt = cnt_ref[0] + cnt_ref[1]
    pc = lax.shift_right_logical(cnt, POS_W.bit_length() - 1)
    present = (cnt & (POS_W - 1)) > 0
    neg_mask = present & (pc == 0)

    x = x_ref[...]
    logit = jnp.clip(1.0 / (1.0 + jnp.exp(-x)), EPS, 1.0 - EPS)
    soft = jnp.log1p(jnp.exp(-jnp.abs(x)))
    bce0 = jnp.maximum(x, 0.0) + soft
    bce1 = bce0 - x
    neg = bce0 * ((1.0 - ALPHA) * jnp.sqrt(logit))
    pos = bce1 * (ALPHA * jnp.sqrt(1.0 - logit))

    total = (jnp.sum(jnp.where(neg_mask, neg, 0.0))
             + jnp.sum(pc.astype(jnp.float32) * pos))
    count = jnp.sum(neg_mask.astype(jnp.float32)) + jnp.float32(N_CLICKED)
    o_ref[0, 0] = total / count


_tc_focal = pl.pallas_call(
    _tc_focal_body,
    out_shape=jax.ShapeDtypeStruct((1, 1), jnp.float32),
    in_specs=[
        pl.BlockSpec(memory_space=pltpu.VMEM),
        pl.BlockSpec(memory_space=pltpu.VMEM),
    ],
    out_specs=pl.BlockSpec(memory_space=pltpu.SMEM),
)


def kernel(input, nodes, clicked):
    zeros = jnp.zeros((NPAD,), jnp.int32)
    ones_w = jnp.ones((W,), jnp.int32)
    posw = jnp.full((N_CLICKED,), POS_W, jnp.int32)
    cnt = _sc_scatter(nodes[0], zeros, ones_w, posw, clicked.astype(jnp.int32))
    x_pad = jnp.pad(input, (0, NPAD - N_NODES)).reshape(ROWS, LANES)
    cnt3 = cnt.reshape(NUM_CORES, ROWS, LANES)
    out = _tc_focal(x_pad, cnt3)
    return out[0, 0]


# speedup vs baseline: 177.6453x; 1.1666x over previous
import jax
import jax.numpy as jnp
from jax import lax
from jax.experimental import pallas as pl
from jax.experimental.pallas import tpu as pltpu
from jax.experimental.pallas import tpu_sc as plsc

N_NODES = 100000
N_EDGES = 3200000
N_CLICKED = 64
GAMMA = 0.5
ALPHA = 0.25
EPS = 1e-6

NPAD = 100352
ROWS = 784
LANES = 128
NUM_CORES = 2
NUM_SUBCORES = 16
NW = NUM_CORES * NUM_SUBCORES

TILES_TOTAL = N_EDGES // 128
TILES_BASE = TILES_TOTAL // NW
TILES_EXTRA = TILES_TOTAL - TILES_BASE * NW
WIN_TILES = 32
WIN = WIN_TILES * 128
N_FULL = TILES_BASE // WIN_TILES
MAX_TAIL = TILES_BASE + 1 - N_FULL * WIN_TILES
POS_W = 1 << 24


def _sc_scatter_body(nodes_hbm, clicked_hbm, out_hbm,
                     buf0, buf1, tail_v, clk_v, mask_v, sem0, sem1):
    c = lax.axis_index("c")
    s = lax.axis_index("s")
    wid = c * NUM_SUBCORES + s

    one16 = jnp.ones((16,), jnp.int32)
    zero16 = jnp.zeros((16,), jnp.int32)

    base_t = wid * TILES_BASE + jnp.minimum(wid, TILES_EXTRA)
    n_tail = TILES_BASE - N_FULL * WIN_TILES + jnp.where(wid < TILES_EXTRA, 1, 0)

    def win_src(w):
        return nodes_hbm.at[:, pl.ds((base_t + w * WIN_TILES) * 128, WIN)]

    pltpu.async_copy(win_src(0), buf0, sem0)
    pltpu.async_copy(win_src(1), buf1, sem1)

    def zero_body(i, carry):
        for u in range(16):
            mask_v[pl.ds(i * 256 + u * 16, 16)] = zero16
        return carry

    lax.fori_loop(0, NPAD // 256, zero_body, 0)

    def scatter_block(buf):
        def inner(k, carry):
            for u in range(8):
                idx16 = buf[0, pl.ds(k * 128 + u * 16, 16)]
                plsc.store_scatter(mask_v, [idx16], one16)
            return carry
        lax.fori_loop(0, WIN // 128, inner, 0)

    def ring_body(j, carry):
        w0 = 2 * j
        pltpu.make_async_copy(win_src(0), buf0, sem0).wait()
        scatter_block(buf0)

        @pl.when(w0 + 2 < N_FULL)
        def _():
            pltpu.async_copy(win_src(w0 + 2), buf0, sem0)

        pltpu.make_async_copy(win_src(0), buf1, sem1).wait()
        scatter_block(buf1)

        @pl.when(w0 + 3 < N_FULL)
        def _():
            pltpu.async_copy(win_src(w0 + 3), buf1, sem1)

        return carry

    lax.fori_loop(0, N_FULL // 2, ring_body, 0)

    def tail_body(k, carry):
        col = (base_t + N_FULL * WIN_TILES + k) * 128
        pltpu.sync_copy(nodes_hbm.at[:, pl.ds(col, 128)], tail_v)
        for u in range(8):
            idx16 = tail_v[0, pl.ds(u * 16, 16)]
            plsc.store_scatter(mask_v, [idx16], one16)
        return carry

    lax.fori_loop(0, n_tail, tail_body, 0)

    @pl.when(wid == 0)
    def _clicked():
        pltpu.sync_copy(clicked_hbm, clk_v)
        posw16 = jnp.full((16,), POS_W, jnp.int32)
        lanes = lax.broadcasted_iota(jnp.int32, (16,), 0)
        for g in range(N_CLICKED // 16):
            idx16 = clk_v[pl.ds(g * 16, 16)]
            for lane in range(16):
                plsc.addupdate_scatter(mask_v, [idx16], posw16,
                                       mask=lanes == lane)

    pltpu.sync_copy(mask_v, out_hbm.at[wid])


_sc_scatter = pl.kernel(
    _sc_scatter_body,
    out_type=jax.ShapeDtypeStruct((NW, NPAD), jnp.int32),
    mesh=plsc.VectorSubcoreMesh(core_axis_name="c", subcore_axis_name="s"),
    compiler_params=pltpu.CompilerParams(needs_layout_passes=False),
    scratch_types=[
        pltpu.VMEM((2, WIN), jnp.int32),
        pltpu.VMEM((2, WIN), jnp.int32),
        pltpu.VMEM((2, 128), jnp.int32),
        pltpu.VMEM((N_CLICKED,), jnp.int32),
        pltpu.VMEM((NPAD,), jnp.int32),
        pltpu.SemaphoreType.DMA,
        pltpu.SemaphoreType.DMA,
    ],
)


def _tc_focal_body(x_ref, cnt_ref, o_ref):
    cnt = cnt_ref[0]
    for t in range(1, NW):
        cnt = cnt + cnt_ref[t]
    pc = lax.shift_right_logical(cnt, POS_W.bit_length() - 1)
    present = (cnt & (POS_W - 1)) > 0
    neg_mask = present & (pc == 0)

    x = x_ref[...]
    logit = jnp.clip(1.0 / (1.0 + jnp.exp(-x)), EPS, 1.0 - EPS)
    soft = jnp.log1p(jnp.exp(-jnp.abs(x)))
    bce0 = jnp.maximum(x, 0.0) + soft
    bce1 = bce0 - x
    neg = bce0 * ((1.0 - ALPHA) * jnp.sqrt(logit))
    pos = bce1 * (ALPHA * jnp.sqrt(1.0 - logit))

    total = (jnp.sum(jnp.where(neg_mask, neg, 0.0))
             + jnp.sum(pc.astype(jnp.float32) * pos))
    count = jnp.sum(neg_mask.astype(jnp.float32)) + jnp.float32(N_CLICKED)
    o_ref[0, 0] = total / count


_tc_focal = pl.pallas_call(
    _tc_focal_body,
    out_shape=jax.ShapeDtypeStruct((1, 1), jnp.float32),
    in_specs=[
        pl.BlockSpec(memory_space=pltpu.VMEM),
        pl.BlockSpec(memory_space=pltpu.VMEM),
    ],
    out_specs=pl.BlockSpec(memory_space=pltpu.SMEM),
)


def kernel(input, nodes, clicked):
    cnt = _sc_scatter(nodes, clicked.astype(jnp.int32))
    x_pad = jnp.pad(input, (0, NPAD - N_NODES)).reshape(ROWS, LANES)
    cnt3 = cnt.reshape(NW, ROWS, LANES)
    out = _tc_focal(x_pad, cnt3)
    return out[0, 0]


# speedup vs baseline: 211.9646x; 1.1932x over previous
import jax
import jax.numpy as jnp
from jax import lax
from jax.experimental import pallas as pl
from jax.experimental.pallas import tpu as pltpu
from jax.experimental.pallas import tpu_sc as plsc

N_NODES = 100000
N_EDGES = 3200000
N_CLICKED = 64
GAMMA = 0.5
ALPHA = 0.25
EPS = 1e-6

NPAD = 100352
ROWS = 784
LANES = 128
NUM_CORES = 2
NUM_SUBCORES = 16
NW = NUM_CORES * NUM_SUBCORES

TILES_TOTAL = N_EDGES // 128
TILES_BASE = TILES_TOTAL // NW
TILES_EXTRA = TILES_TOTAL - TILES_BASE * NW
WIN_TILES = 32
WIN = WIN_TILES * 128
N_FULL = TILES_BASE // WIN_TILES
MAX_TAIL = TILES_BASE + 1 - N_FULL * WIN_TILES
POS_W = 1 << 24

MERGE_CHUNK = 112
N_MERGE = ROWS // MERGE_CHUNK


def _sc_scatter_body(nodes_hbm, clicked_hbm, out_hbm,
                     buf0, buf1, tail_v, clk_v, ridx_v, mask_v, cnt_s,
                     sem0, sem1):
    c = lax.axis_index("c")
    s = lax.axis_index("s")
    wid = c * NUM_SUBCORES + s

    one16 = jnp.ones((16,), jnp.int32)
    zero16 = jnp.zeros((16,), jnp.int32)
    lanes16 = lax.broadcasted_iota(jnp.int32, (16,), 0)

    base_t = wid * TILES_BASE + jnp.minimum(wid, TILES_EXTRA)
    n_tail = TILES_BASE - N_FULL * WIN_TILES + jnp.where(wid < TILES_EXTRA, 1, 0)

    def win_src(w):
        return nodes_hbm.at[:, pl.ds((base_t + w * WIN_TILES) * 128, WIN)]

    pltpu.async_copy(win_src(0), buf0, sem0)
    pltpu.async_copy(win_src(1), buf1, sem1)

    def zero_body(i, carry):
        for u in range(LANES // 16):
            mask_v[i, pl.ds(u * 16, 16)] = zero16
        return carry

    lax.fori_loop(0, ROWS, zero_body, 0)

    for j in range(N_MERGE):
        for u in range(MERGE_CHUNK // 16):
            ridx_v[j, pl.ds(u * 16, 16)] = lanes16 + (j * MERGE_CHUNK + u * 16)

    @pl.when(s == 0)
    def _init_shared():
        pltpu.sync_copy(mask_v, cnt_s)

    plsc.subcore_barrier()

    def scatter16(idx16):
        r16 = lax.shift_right_logical(idx16, 7)
        c16 = lax.bitwise_and(idx16, 127)
        plsc.store_scatter(mask_v, [r16, c16], one16)

    def scatter_block(buf):
        def inner(k, carry):
            for u in range(8):
                scatter16(buf[0, pl.ds(k * 128 + u * 16, 16)])
            return carry
        lax.fori_loop(0, WIN // 128, inner, 0)

    def ring_body(j, carry):
        w0 = 2 * j
        pltpu.make_async_copy(win_src(0), buf0, sem0).wait()
        scatter_block(buf0)

        @pl.when(w0 + 2 < N_FULL)
        def _():
            pltpu.async_copy(win_src(w0 + 2), buf0, sem0)

        pltpu.make_async_copy(win_src(0), buf1, sem1).wait()
        scatter_block(buf1)

        @pl.when(w0 + 3 < N_FULL)
        def _():
            pltpu.async_copy(win_src(w0 + 3), buf1, sem1)

        return carry

    lax.fori_loop(0, N_FULL // 2, ring_body, 0)

    def tail_body(k, carry):
        col = (base_t + N_FULL * WIN_TILES + k) * 128
        pltpu.sync_copy(nodes_hbm.at[:, pl.ds(col, 128)], tail_v)
        for u in range(8):
            scatter16(tail_v[0, pl.ds(u * 16, 16)])
        return carry

    lax.fori_loop(0, n_tail, tail_body, 0)

    @pl.when(wid == 0)
    def _clicked():
        pltpu.sync_copy(clicked_hbm, clk_v)
        posw16 = jnp.full((16,), POS_W, jnp.int32)
        for g in range(N_CLICKED // 16):
            idx16 = clk_v[pl.ds(g * 16, 16)]
            r16 = lax.shift_right_logical(idx16, 7)
            c16 = lax.bitwise_and(idx16, 127)
            for lane in range(16):
                plsc.addupdate_scatter(mask_v, [r16, c16], posw16,
                                       mask=lanes16 == lane)

    for j in range(N_MERGE):
        pltpu.sync_copy(mask_v.at[pl.ds(j * MERGE_CHUNK, MERGE_CHUNK)],
                        cnt_s.at[ridx_v.at[j]], add=True)

    plsc.subcore_barrier()

    @pl.when(s == 0)
    def _flush():
        pltpu.sync_copy(cnt_s, out_hbm.at[c])


_sc_scatter = pl.kernel(
    _sc_scatter_body,
    out_type=jax.ShapeDtypeStruct((NUM_CORES, ROWS, LANES), jnp.int32),
    mesh=plsc.VectorSubcoreMesh(core_axis_name="c", subcore_axis_name="s"),
    compiler_params=pltpu.CompilerParams(needs_layout_passes=False),
    scratch_types=[
        pltpu.VMEM((2, WIN), jnp.int32),
        pltpu.VMEM((2, WIN), jnp.int32),
        pltpu.VMEM((2, 128), jnp.int32),
        pltpu.VMEM((N_CLICKED,), jnp.int32),
        pltpu.VMEM((N_MERGE, MERGE_CHUNK), jnp.int32),
        pltpu.VMEM((ROWS, LANES), jnp.int32),
        pltpu.VMEM_SHARED((ROWS, LANES), jnp.int32),
        pltpu.SemaphoreType.DMA,
        pltpu.SemaphoreType.DMA,
    ],
)


def _tc_focal_body(x_ref, cnt_ref, o_ref):
    cnt = cnt_ref[0]
    for t in range(1, NUM_CORES):
        cnt = cnt + cnt_ref[t]
    pc = lax.shift_right_logical(cnt, POS_W.bit_length() - 1)
    present = (cnt & (POS_W - 1)) > 0
    neg_mask = present & (pc == 0)

    x = x_ref[...]
    logit = jnp.clip(1.0 / (1.0 + jnp.exp(-x)), EPS, 1.0 - EPS)
    soft = jnp.log1p(jnp.exp(-jnp.abs(x)))
    bce0 = jnp.maximum(x, 0.0) + soft
    bce1 = bce0 - x
    neg = bce0 * ((1.0 - ALPHA) * jnp.sqrt(logit))
    pos = bce1 * (ALPHA * jnp.sqrt(1.0 - logit))

    total = (jnp.sum(jnp.where(neg_mask, neg, 0.0))
             + jnp.sum(pc.astype(jnp.float32) * pos))
    count = jnp.sum(neg_mask.astype(jnp.float32)) + jnp.float32(N_CLICKED)
    o_ref[0, 0] = total / count


_tc_focal = pl.pallas_call(
    _tc_focal_body,
    out_shape=jax.ShapeDtypeStruct((1, 1), jnp.float32),
    in_specs=[
        pl.BlockSpec(memory_space=pltpu.VMEM),
        pl.BlockSpec(memory_space=pltpu.VMEM),
    ],
    out_specs=pl.BlockSpec(memory_space=pltpu.SMEM),
)


def kernel(input, nodes, clicked):
    cnt = _sc_scatter(nodes, clicked.astype(jnp.int32))
    x_pad = jnp.pad(input, (0, NPAD - N_NODES)).reshape(ROWS, LANES)
    out = _tc_focal(x_pad, cnt)
    return out[0, 0]


# speedup vs baseline: 228.7080x; 1.0790x over previous
import jax
import jax.numpy as jnp
from jax import lax
from jax.experimental import pallas as pl
from jax.experimental.pallas import tpu as pltpu
from jax.experimental.pallas import tpu_sc as plsc

N_NODES = 100000
N_EDGES = 3200000
N_CLICKED = 64
GAMMA = 0.5
ALPHA = 0.25
EPS = 1e-6

NPAD = 100352
ROWS = 784
LANES = 128
NUM_CORES = 2
NUM_SUBCORES = 16
NW = NUM_CORES * NUM_SUBCORES

TILES_TOTAL = N_EDGES // 128
TILES_BASE = TILES_TOTAL // NW
TILES_EXTRA = TILES_TOTAL - TILES_BASE * NW
WIN_TILES = 39
WIN = WIN_TILES * 128
N_FULL = TILES_BASE // WIN_TILES
MAX_TAIL = TILES_BASE + 1 - N_FULL * WIN_TILES
POS_W = 1 << 24

MERGE_CHUNK = 112
N_MERGE = ROWS // MERGE_CHUNK


def _sc_scatter_body(nodes_hbm, clicked_hbm, out_hbm,
                     buf0, buf1, tail_v, clk_v, ridx_v, mask_v, cnt_s,
                     sem0, sem1):
    c = lax.axis_index("c")
    s = lax.axis_index("s")
    wid = c * NUM_SUBCORES + s

    one16 = jnp.ones((16,), jnp.int32)
    zero16 = jnp.zeros((16,), jnp.int32)
    lanes16 = lax.broadcasted_iota(jnp.int32, (16,), 0)

    base_t = wid * TILES_BASE + jnp.minimum(wid, TILES_EXTRA)
    n_tail = TILES_BASE - N_FULL * WIN_TILES + jnp.where(wid < TILES_EXTRA, 1, 0)

    def win_src(w):
        return nodes_hbm.at[:, pl.ds((base_t + w * WIN_TILES) * 128, WIN)]

    pltpu.async_copy(win_src(0), buf0, sem0)
    pltpu.async_copy(win_src(1), buf1, sem1)

    def zero_body(i, carry):
        for r in range(8):
            for u in range(LANES // 16):
                mask_v[i * 8 + r, pl.ds(u * 16, 16)] = zero16
        return carry

    lax.fori_loop(0, ROWS // 8, zero_body, 0)

    for j in range(N_MERGE):
        for u in range(MERGE_CHUNK // 16):
            ridx_v[j, pl.ds(u * 16, 16)] = lanes16 + (j * MERGE_CHUNK + u * 16)

    @pl.when(s == 0)
    def _init_shared():
        pltpu.sync_copy(mask_v, cnt_s)

    plsc.subcore_barrier()

    def scatter16(idx16):
        r16 = lax.shift_right_logical(idx16, 7)
        c16 = lax.bitwise_and(idx16, 127)
        plsc.store_scatter(mask_v, [r16, c16], one16)

    def scatter_block(buf):
        def inner(k, carry):
            for u in range(8):
                scatter16(buf[0, pl.ds(k * 128 + u * 16, 16)])
            return carry
        lax.fori_loop(0, WIN // 128, inner, 0)

    def ring_body(j, carry):
        w0 = 2 * j
        pltpu.make_async_copy(win_src(0), buf0, sem0).wait()
        scatter_block(buf0)

        @pl.when(w0 + 2 < N_FULL)
        def _():
            pltpu.async_copy(win_src(w0 + 2), buf0, sem0)

        pltpu.make_async_copy(win_src(0), buf1, sem1).wait()
        scatter_block(buf1)

        @pl.when(w0 + 3 < N_FULL)
        def _():
            pltpu.async_copy(win_src(w0 + 3), buf1, sem1)

        return carry

    lax.fori_loop(0, N_FULL // 2, ring_body, 0)

    def tail_win(k):
        col = (base_t + N_FULL * WIN_TILES + k) * 128
        pltpu.sync_copy(nodes_hbm.at[:, pl.ds(col, 128)], tail_v)
        for u in range(8):
            scatter16(tail_v[0, pl.ds(u * 16, 16)])

    tail_win(0)

    @pl.when(n_tail > 1)
    def _tail2():
        tail_win(1)

    @pl.when(wid == 0)
    def _clicked():
        pltpu.sync_copy(clicked_hbm, clk_v)
        posw16 = jnp.full((16,), POS_W, jnp.int32)
        for g in range(N_CLICKED // 16):
            idx16 = clk_v[pl.ds(g * 16, 16)]
            r16 = lax.shift_right_logical(idx16, 7)
            c16 = lax.bitwise_and(idx16, 127)
            for lane in range(16):
                plsc.addupdate_scatter(mask_v, [r16, c16], posw16,
                                       mask=lanes16 == lane)

    for j in range(N_MERGE):
        pltpu.sync_copy(mask_v.at[pl.ds(j * MERGE_CHUNK, MERGE_CHUNK)],
                        cnt_s.at[ridx_v.at[j]], add=True)

    plsc.subcore_barrier()

    @pl.when(s == 0)
    def _flush():
        pltpu.sync_copy(cnt_s, out_hbm.at[c])


_sc_scatter = pl.kernel(
    _sc_scatter_body,
    out_type=jax.ShapeDtypeStruct((NUM_CORES, ROWS, LANES), jnp.int32),
    mesh=plsc.VectorSubcoreMesh(core_axis_name="c", subcore_axis_name="s"),
    compiler_params=pltpu.CompilerParams(needs_layout_passes=False),
    scratch_types=[
        pltpu.VMEM((2, WIN), jnp.int32),
        pltpu.VMEM((2, WIN), jnp.int32),
        pltpu.VMEM((2, 128), jnp.int32),
        pltpu.VMEM((N_CLICKED,), jnp.int32),
        pltpu.VMEM((N_MERGE, MERGE_CHUNK), jnp.int32),
        pltpu.VMEM((ROWS, LANES), jnp.int32),
        pltpu.VMEM_SHARED((ROWS, LANES), jnp.int32),
        pltpu.SemaphoreType.DMA,
        pltpu.SemaphoreType.DMA,
    ],
)


def _tc_focal_body(x_ref, cnt_ref, o_ref):
    cnt = cnt_ref[0]
    for t in range(1, NUM_CORES):
        cnt = cnt + cnt_ref[t]
    pc = lax.shift_right_logical(cnt, POS_W.bit_length() - 1)
    present = (cnt & (POS_W - 1)) > 0
    neg_mask = present & (pc == 0)

    x = x_ref[...]
    logit = jnp.clip(1.0 / (1.0 + jnp.exp(-x)), EPS, 1.0 - EPS)
    soft = jnp.log1p(jnp.exp(-jnp.abs(x)))
    bce0 = jnp.maximum(x, 0.0) + soft
    bce1 = bce0 - x
    neg = bce0 * ((1.0 - ALPHA) * jnp.sqrt(logit))
    pos = bce1 * (ALPHA * jnp.sqrt(1.0 - logit))

    total = (jnp.sum(jnp.where(neg_mask, neg, 0.0))
             + jnp.sum(pc.astype(jnp.float32) * pos))
    count = jnp.sum(neg_mask.astype(jnp.float32)) + jnp.float32(N_CLICKED)
    o_ref[0, 0] = total / count


_tc_focal = pl.pallas_call(
    _tc_focal_body,
    out_shape=jax.ShapeDtypeStruct((1, 1), jnp.float32),
    in_specs=[
        pl.BlockSpec(memory_space=pltpu.VMEM),
        pl.BlockSpec(memory_space=pltpu.VMEM),
    ],
    out_specs=pl.BlockSpec(memory_space=pltpu.SMEM),
)


def kernel(input, nodes, clicked):
    cnt = _sc_scatter(nodes, clicked.astype(jnp.int32))
    x_pad = jnp.pad(input, (0, NPAD - N_NODES)).reshape(ROWS, LANES)
    out = _tc_focal(x_pad, cnt)
    return out[0, 0]


# speedup vs baseline: 321.3263x; 1.4050x over previous
import jax
import jax.numpy as jnp
from jax import lax
from jax.experimental import pallas as pl
from jax.experimental.pallas import tpu as pltpu
from jax.experimental.pallas import tpu_sc as plsc

N_NODES = 100000
N_EDGES = 3200000
N_CLICKED = 64
GAMMA = 0.5
ALPHA = 0.25
EPS = 1e-6

NPAD = 100352
ROWS = 784
LANES = 128
NUM_CORES = 2
NUM_SUBCORES = 16
NW = NUM_CORES * NUM_SUBCORES

TILES_TOTAL = N_EDGES // 128
TILES_BASE = TILES_TOTAL // NW
TILES_EXTRA = TILES_TOTAL - TILES_BASE * NW
WIN_TILES = 39
WIN = WIN_TILES * 128
N_FULL = TILES_BASE // WIN_TILES
MAX_TAIL = TILES_BASE + 1 - N_FULL * WIN_TILES
POS_W = 1 << 24

MERGE_CHUNK = 112
N_MERGE = ROWS // MERGE_CHUNK


def _sc_scatter_body(nodes_hbm, clicked_hbm, out_hbm,
                     buf0, buf1, tail_v, clk_v, ridx_v, mask_v, cnt_s,
                     sem0, sem1):
    c = lax.axis_index("c")
    s = lax.axis_index("s")
    wid = c * NUM_SUBCORES + s

    one16 = jnp.ones((16,), jnp.int32)
    zero16 = jnp.zeros((16,), jnp.int32)
    lanes16 = lax.broadcasted_iota(jnp.int32, (16,), 0)

    base_t = wid * TILES_BASE + jnp.minimum(wid, TILES_EXTRA)
    n_tail = TILES_BASE - N_FULL * WIN_TILES + jnp.where(wid < TILES_EXTRA, 1, 0)

    def win_src(w):
        return nodes_hbm.at[:, pl.ds((base_t + w * WIN_TILES) * 128, WIN)]

    pltpu.async_copy(win_src(0), buf0, sem0)
    pltpu.async_copy(win_src(1), buf1, sem1)

    @plsc.parallel_loop(0, ROWS, 1, unroll=8)
    def _zero(i):
        for u in range(LANES // 16):
            mask_v[i, pl.ds(u * 16, 16)] = zero16

    for j in range(N_MERGE):
        for u in range(MERGE_CHUNK // 16):
            ridx_v[j, pl.ds(u * 16, 16)] = lanes16 + (j * MERGE_CHUNK + u * 16)

    @pl.when(s == 0)
    def _init_shared():
        pltpu.sync_copy(mask_v, cnt_s)

    plsc.subcore_barrier()

    def scatter16(idx16):
        r16 = lax.shift_right_logical(idx16, 7)
        c16 = lax.bitwise_and(idx16, 127)
        plsc.store_scatter(mask_v, [r16, c16], one16)

    def scatter_block(buf):
        @plsc.parallel_loop(0, WIN, 16, unroll=8)
        def _(k):
            scatter16(buf[0, pl.ds(k, 16)])

    def ring_body(j, carry):
        w0 = 2 * j
        pltpu.make_async_copy(win_src(0), buf0, sem0).wait()
        scatter_block(buf0)

        @pl.when(w0 + 2 < N_FULL)
        def _():
            pltpu.async_copy(win_src(w0 + 2), buf0, sem0)

        pltpu.make_async_copy(win_src(0), buf1, sem1).wait()
        scatter_block(buf1)

        @pl.when(w0 + 3 < N_FULL)
        def _():
            pltpu.async_copy(win_src(w0 + 3), buf1, sem1)

        return carry

    lax.fori_loop(0, N_FULL // 2, ring_body, 0)

    def tail_win(k):
        col = (base_t + N_FULL * WIN_TILES + k) * 128
        pltpu.sync_copy(nodes_hbm.at[:, pl.ds(col, 128)], tail_v)
        for u in range(8):
            scatter16(tail_v[0, pl.ds(u * 16, 16)])

    tail_win(0)

    @pl.when(n_tail > 1)
    def _tail2():
        tail_win(1)

    @pl.when(wid == 0)
    def _clicked():
        pltpu.sync_copy(clicked_hbm, clk_v)
        posw16 = jnp.full((16,), POS_W, jnp.int32)
        for g in range(N_CLICKED // 16):
            idx16 = clk_v[pl.ds(g * 16, 16)]
            r16 = lax.shift_right_logical(idx16, 7)
            c16 = lax.bitwise_and(idx16, 127)
            for lane in range(16):
                plsc.addupdate_scatter(mask_v, [r16, c16], posw16,
                                       mask=lanes16 == lane)

    for j in range(N_MERGE):
        pltpu.sync_copy(mask_v.at[pl.ds(j * MERGE_CHUNK, MERGE_CHUNK)],
                        cnt_s.at[ridx_v.at[j]], add=True)

    plsc.subcore_barrier()

    @pl.when(s == 0)
    def _flush():
        pltpu.sync_copy(cnt_s, out_hbm.at[c])


_sc_scatter = pl.kernel(
    _sc_scatter_body,
    out_type=jax.ShapeDtypeStruct((NUM_CORES, ROWS, LANES), jnp.int32),
    mesh=plsc.VectorSubcoreMesh(core_axis_name="c", subcore_axis_name="s"),
    compiler_params=pltpu.CompilerParams(needs_layout_passes=False),
    scratch_types=[
        pltpu.VMEM((2, WIN), jnp.int32),
        pltpu.VMEM((2, WIN), jnp.int32),
        pltpu.VMEM((2, 128), jnp.int32),
        pltpu.VMEM((N_CLICKED,), jnp.int32),
        pltpu.VMEM((N_MERGE, MERGE_CHUNK), jnp.int32),
        pltpu.VMEM((ROWS, LANES), jnp.int32),
        pltpu.VMEM_SHARED((ROWS, LANES), jnp.int32),
        pltpu.SemaphoreType.DMA,
        pltpu.SemaphoreType.DMA,
    ],
)


def _tc_focal_body(x_ref, cnt_ref, o_ref):
    cnt = cnt_ref[0]
    for t in range(1, NUM_CORES):
        cnt = cnt + cnt_ref[t]
    pc = lax.shift_right_logical(cnt, POS_W.bit_length() - 1)
    present = (cnt & (POS_W - 1)) > 0
    neg_mask = present & (pc == 0)

    x = x_ref[...]
    logit = jnp.clip(1.0 / (1.0 + jnp.exp(-x)), EPS, 1.0 - EPS)
    soft = jnp.log1p(jnp.exp(-jnp.abs(x)))
    bce0 = jnp.maximum(x, 0.0) + soft
    bce1 = bce0 - x
    neg = bce0 * ((1.0 - ALPHA) * jnp.sqrt(logit))
    pos = bce1 * (ALPHA * jnp.sqrt(1.0 - logit))

    total = (jnp.sum(jnp.where(neg_mask, neg, 0.0))
             + jnp.sum(pc.astype(jnp.float32) * pos))
    count = jnp.sum(neg_mask.astype(jnp.float32)) + jnp.float32(N_CLICKED)
    o_ref[0, 0] = total / count


_tc_focal = pl.pallas_call(
    _tc_focal_body,
    out_shape=jax.ShapeDtypeStruct((1, 1), jnp.float32),
    in_specs=[
        pl.BlockSpec(memory_space=pltpu.VMEM),
        pl.BlockSpec(memory_space=pltpu.VMEM),
    ],
    out_specs=pl.BlockSpec(memory_space=pltpu.SMEM),
)


def kernel(input, nodes, clicked):
    cnt = _sc_scatter(nodes, clicked.astype(jnp.int32))
    x_pad = jnp.pad(input, (0, NPAD - N_NODES)).reshape(ROWS, LANES)
    out = _tc_focal(x_pad, cnt)
    return out[0, 0]


# speedup vs baseline: 325.4250x; 1.0128x over previous
import jax
import jax.numpy as jnp
from jax import lax
from jax.experimental import pallas as pl
from jax.experimental.pallas import tpu as pltpu
from jax.experimental.pallas import tpu_sc as plsc

N_NODES = 100000
N_EDGES = 3200000
N_CLICKED = 64
GAMMA = 0.5
ALPHA = 0.25
EPS = 1e-6

NPAD = 100352
ROWS = 784
LANES = 128
NUM_CORES = 2
NUM_SUBCORES = 16
NW = NUM_CORES * NUM_SUBCORES

TILES_TOTAL = N_EDGES // 128
TILES_BASE = TILES_TOTAL // NW
TILES_EXTRA = TILES_TOTAL - TILES_BASE * NW
WIN_TILES = 39
WIN = WIN_TILES * 128
N_FULL = TILES_BASE // WIN_TILES
MAX_TAIL = TILES_BASE + 1 - N_FULL * WIN_TILES
POS_W = 1 << 24

MERGE_CHUNK = 112
N_MERGE = ROWS // MERGE_CHUNK


def _sc_scatter_body(nodes_hbm, clicked_hbm, out_hbm,
                     buf0, buf1, tail_v, clk_v, ridx_v, mask_v, cnt_s,
                     sem0, sem1):
    c = lax.axis_index("c")
    s = lax.axis_index("s")
    wid = c * NUM_SUBCORES + s

    one16 = jnp.ones((16,), jnp.int32)
    zero16 = jnp.zeros((16,), jnp.int32)
    lanes16 = lax.broadcasted_iota(jnp.int32, (16,), 0)

    base_t = wid * TILES_BASE + jnp.minimum(wid, TILES_EXTRA)
    n_tail = TILES_BASE - N_FULL * WIN_TILES + jnp.where(wid < TILES_EXTRA, 1, 0)

    def win_src(w):
        return nodes_hbm.at[:, pl.ds((base_t + w * WIN_TILES) * 128, WIN)]

    pltpu.async_copy(win_src(0), buf0, sem0)
    pltpu.async_copy(win_src(1), buf1, sem1)

    @plsc.parallel_loop(0, ROWS, 1, unroll=8)
    def _zero(i):
        for u in range(LANES // 16):
            mask_v[i, pl.ds(u * 16, 16)] = zero16

    for j in range(N_MERGE):
        for u in range(MERGE_CHUNK // 16):
            ridx_v[j, pl.ds(u * 16, 16)] = lanes16 + (j * MERGE_CHUNK + u * 16)

    @pl.when(s == 0)
    def _init_shared():
        pltpu.sync_copy(mask_v, cnt_s)

    plsc.subcore_barrier()

    def scatter16(idx16):
        r16 = lax.shift_right_logical(idx16, 7)
        c16 = lax.bitwise_and(idx16, 127)
        plsc.store_scatter(mask_v, [r16, c16], one16)

    def scatter_block(buf):
        @plsc.parallel_loop(0, WIN, 16, unroll=8)
        def _(k):
            scatter16(buf[0, pl.ds(k, 16)])

    def ring_body(j, carry):
        w0 = 2 * j
        pltpu.make_async_copy(win_src(0), buf0, sem0).wait()
        scatter_block(buf0)

        @pl.when(w0 + 2 < N_FULL)
        def _():
            pltpu.async_copy(win_src(w0 + 2), buf0, sem0)

        pltpu.make_async_copy(win_src(0), buf1, sem1).wait()
        scatter_block(buf1)

        @pl.when(w0 + 3 < N_FULL)
        def _():
            pltpu.async_copy(win_src(w0 + 3), buf1, sem1)

        return carry

    lax.fori_loop(0, N_FULL // 2, ring_body, 0)

    def tail_win(k):
        col = (base_t + N_FULL * WIN_TILES + k) * 128
        pltpu.sync_copy(nodes_hbm.at[:, pl.ds(col, 128)], tail_v)
        for u in range(8):
            scatter16(tail_v[0, pl.ds(u * 16, 16)])

    tail_win(0)

    @pl.when(n_tail > 1)
    def _tail2():
        tail_win(1)

    @pl.when(wid == 0)
    def _clicked():
        pltpu.sync_copy(clicked_hbm, clk_v)
        posw16 = jnp.full((16,), POS_W, jnp.int32)
        for g in range(N_CLICKED // 16):
            idx16 = clk_v[pl.ds(g * 16, 16)]
            r16 = lax.shift_right_logical(idx16, 7)
            c16 = lax.bitwise_and(idx16, 127)
            for lane in range(16):
                plsc.addupdate_scatter(mask_v, [r16, c16], posw16,
                                       mask=lanes16 == lane)

    for j in range(N_MERGE):
        pltpu.sync_copy(mask_v.at[pl.ds(j * MERGE_CHUNK, MERGE_CHUNK)],
                        cnt_s.at[ridx_v.at[j]], add=True)

    plsc.subcore_barrier()

    @pl.when(s == 0)
    def _flush():
        pltpu.sync_copy(cnt_s, out_hbm.at[c])


_sc_scatter = pl.kernel(
    _sc_scatter_body,
    out_type=jax.ShapeDtypeStruct((NUM_CORES, ROWS, LANES), jnp.int32),
    mesh=plsc.VectorSubcoreMesh(core_axis_name="c", subcore_axis_name="s"),
    compiler_params=pltpu.CompilerParams(needs_layout_passes=False),
    scratch_types=[
        pltpu.VMEM((2, WIN), jnp.int32),
        pltpu.VMEM((2, WIN), jnp.int32),
        pltpu.VMEM((2, 128), jnp.int32),
        pltpu.VMEM((N_CLICKED,), jnp.int32),
        pltpu.VMEM((N_MERGE, MERGE_CHUNK), jnp.int32),
        pltpu.VMEM((ROWS, LANES), jnp.int32),
        pltpu.VMEM_SHARED((ROWS, LANES), jnp.int32),
        pltpu.SemaphoreType.DMA,
        pltpu.SemaphoreType.DMA,
    ],
)


def _tc_pre_body(x_ref, neg_ref, pos_ref):
    x = x_ref[...]
    logit = jnp.clip(1.0 / (1.0 + jnp.exp(-x)), EPS, 1.0 - EPS)
    soft = jnp.log1p(jnp.exp(-jnp.abs(x)))
    bce0 = jnp.maximum(x, 0.0) + soft
    bce1 = bce0 - x
    neg_ref[...] = bce0 * ((1.0 - ALPHA) * jnp.sqrt(logit))
    pos_ref[...] = bce1 * (ALPHA * jnp.sqrt(1.0 - logit))


_tc_pre = pl.pallas_call(
    _tc_pre_body,
    out_shape=[
        jax.ShapeDtypeStruct((ROWS, LANES), jnp.float32),
        jax.ShapeDtypeStruct((ROWS, LANES), jnp.float32),
    ],
    in_specs=[pl.BlockSpec(memory_space=pltpu.VMEM)],
    out_specs=[
        pl.BlockSpec(memory_space=pltpu.VMEM),
        pl.BlockSpec(memory_space=pltpu.VMEM),
    ],
)


def _tc_reduce_body(cnt_ref, neg_ref, pos_ref, o_ref):
    cnt = cnt_ref[0]
    for t in range(1, NUM_CORES):
        cnt = cnt + cnt_ref[t]
    pc = lax.shift_right_logical(cnt, POS_W.bit_length() - 1)
    present = (cnt & (POS_W - 1)) > 0
    neg_mask = present & (pc == 0)

    total = (jnp.sum(jnp.where(neg_mask, neg_ref[...], 0.0))
             + jnp.sum(pc.astype(jnp.float32) * pos_ref[...]))
    count = jnp.sum(neg_mask.astype(jnp.float32)) + jnp.float32(N_CLICKED)
    o_ref[0, 0] = total / count


_tc_reduce = pl.pallas_call(
    _tc_reduce_body,
    out_shape=jax.ShapeDtypeStruct((1, 1), jnp.float32),
    in_specs=[
        pl.BlockSpec(memory_space=pltpu.VMEM),
        pl.BlockSpec(memory_space=pltpu.VMEM),
        pl.BlockSpec(memory_space=pltpu.VMEM),
    ],
    out_specs=pl.BlockSpec(memory_space=pltpu.SMEM),
)


def kernel(input, nodes, clicked):
    cnt = _sc_scatter(nodes, clicked.astype(jnp.int32))
    x_pad = jnp.pad(input, (0, NPAD - N_NODES)).reshape(ROWS, LANES)
    neg, pos = _tc_pre(x_pad)
    out = _tc_reduce(cnt, neg, pos)
    return out[0, 0]
